# Initial kernel scaffold; baseline (speedup 1.0000x reference)
import jax
import jax.numpy as jnp
from jax.experimental import pallas as pl


def kernel(x, gate_w, gate_b, fc1_w, fc1_b, fc2_w, fc2_b):
    raise NotImplementedError("write your pallas kernel here")


# speedup vs baseline: 4.5639x; 4.5639x over previous
import functools

import jax
import jax.numpy as jnp
from jax import lax
from jax.experimental import pallas as pl
from jax.experimental.pallas import tpu as pltpu
from jax.experimental.pallas import tpu_sc as plsc

E = 64
TOPK = 2
H = 768
F = 1024
N = 4096
A = N * TOPK
T = 128
P = A + E * T
NT = P // T

NC = 2
NS = 16
NW = NC * NS

_SQRT_HALF = 0.7071067811865476


def _gelu_exact(v):
    return 0.5 * v * (1.0 + lax.erf(v * _SQRT_HALF))


_RB = 512


def _router_body(x_ref, gw_ref, gb_ref, idx_ref, rw_ref):
    x = x_ref[...]
    logits = lax.dot_general(x, gw_ref[...], (((1,), (1,)), ((), ())),
                             preferred_element_type=jnp.float32)
    logits = logits + gb_ref[...][None, :]
    ids = lax.broadcasted_iota(jnp.int32, logits.shape, 1)
    neg = jnp.float32(jnp.finfo(jnp.float32).min)
    m1 = jnp.max(logits, axis=1, keepdims=True)
    a1 = jnp.min(jnp.where(logits == m1, ids, E), axis=1, keepdims=True)
    l2 = jnp.where(ids == a1, neg, logits)
    m2 = jnp.max(l2, axis=1, keepdims=True)
    a2 = jnp.min(jnp.where(l2 == m2, ids, E), axis=1, keepdims=True)
    t = jnp.exp(m2 - m1)
    w1 = 1.0 / (1.0 + t)
    idx_ref[...] = jnp.concatenate([a1, a2], axis=1)
    rw_ref[...] = jnp.concatenate([w1, t * w1], axis=1)


def _router(xf, gate_w, gate_b):
    return pl.pallas_call(
        _router_body,
        grid=(N // _RB,),
        in_specs=[
            pl.BlockSpec((_RB, H), lambda i: (i, 0)),
            pl.BlockSpec((E, H), lambda i: (0, 0)),
            pl.BlockSpec((E,), lambda i: (0,)),
        ],
        out_specs=[
            pl.BlockSpec((_RB, TOPK), lambda i: (i, 0)),
            pl.BlockSpec((_RB, TOPK), lambda i: (i, 0)),
        ],
        out_shape=[
            jax.ShapeDtypeStruct((N, TOPK), jnp.int32),
            jax.ShapeDtypeStruct((N, TOPK), jnp.float32),
        ],
        compiler_params=pltpu.CompilerParams(dimension_semantics=("arbitrary",)),
    )(xf, gate_w, gate_b)


_GCH = 128


@functools.lru_cache(maxsize=None)
def _make_sc_gather():
    mesh = plsc.VectorSubcoreMesh(
        core_axis_name="c", subcore_axis_name="s",
        num_cores=NC, num_subcores=NS)

    @functools.partial(
        pl.kernel,
        out_type=jax.ShapeDtypeStruct((P, H), jnp.float32),
        mesh=mesh,
        scratch_types=[
            pltpu.VMEM((_GCH,), jnp.int32),
            pltpu.VMEM((_GCH, H), jnp.float32),
            pltpu.SemaphoreType.DMA,
        ],
    )
    def gather_k(tok_hbm, x_hbm, out_hbm, idx_v, rows_v, sem):
        wid = lax.axis_index("s") * NC + lax.axis_index("c")
        base = wid * (P // NW)

        def chunk(j, carry):
            off = base + j * _GCH
            pltpu.sync_copy(tok_hbm.at[pl.ds(off, _GCH)], idx_v)
            pltpu.async_copy(x_hbm.at[idx_v], rows_v, sem).wait()
            pltpu.sync_copy(rows_v, out_hbm.at[pl.ds(off, _GCH)])
            return carry

        lax.fori_loop(0, (P // NW) // _GCH, chunk, 0)

    return gather_k


def _sc_gather(token_of_slot, xf):
    return _make_sc_gather()(token_of_slot, xf)


def _ffn_body(er_ref, nr_ref, x_ref, w1_ref, b1_ref, w2_ref, b2_ref, sw_ref,
              o_ref):
    t = pl.program_id(0)

    @pl.when(t < nr_ref[0])
    def _():
        x = x_ref[...]
        h = lax.dot_general(x, w1_ref[0], (((1,), (1,)), ((), ())),
                            preferred_element_type=jnp.float32)
        h = _gelu_exact(h + b1_ref[0])
        o = lax.dot_general(h, w2_ref[0], (((1,), (1,)), ((), ())),
                            preferred_element_type=jnp.float32)
        o_ref[...] = (o + b2_ref[0]) * sw_ref[...]


def _gmm(expert_of_tile, n_active, x_sorted, fc1_w, fc1_b, fc2_w, fc2_b,
         slot_w):
    def rowblk(t, er, nr):
        return (jnp.minimum(t, nr[0] - 1), 0)

    grid_spec = pltpu.PrefetchScalarGridSpec(
        num_scalar_prefetch=2,
        grid=(NT,),
        in_specs=[
            pl.BlockSpec((T, H), rowblk),
            pl.BlockSpec((1, F, H), lambda t, er, nr: (er[t], 0, 0)),
            pl.BlockSpec((1, 1, F), lambda t, er, nr: (er[t], 0, 0)),
            pl.BlockSpec((1, H, F), lambda t, er, nr: (er[t], 0, 0)),
            pl.BlockSpec((1, 1, H), lambda t, er, nr: (er[t], 0, 0)),
            pl.BlockSpec((T, 1), rowblk),
        ],
        out_specs=pl.BlockSpec((T, H), rowblk),
    )
    return pl.pallas_call(
        _ffn_body,
        grid_spec=grid_spec,
        out_shape=jax.ShapeDtypeStruct((P, H), jnp.float32),
        compiler_params=pltpu.CompilerParams(dimension_semantics=("arbitrary",)),
    )(expert_of_tile, n_active, x_sorted, fc1_w, fc1_b.reshape(E, 1, F),
      fc2_w, fc2_b.reshape(E, 1, H), slot_w)


_CT = 32


@functools.lru_cache(maxsize=None)
def _make_sc_combine():
    mesh = plsc.VectorSubcoreMesh(
        core_axis_name="c", subcore_axis_name="s",
        num_cores=NC, num_subcores=NS)

    @functools.partial(
        pl.kernel,
        out_type=jax.ShapeDtypeStruct((N, H), jnp.float32),
        mesh=mesh,
        scratch_types=[
            pltpu.VMEM((2 * _CT,), jnp.int32),
            pltpu.VMEM((2 * _CT, H), jnp.float32),
            pltpu.VMEM((_CT, H), jnp.float32),
            pltpu.SemaphoreType.DMA,
        ],
    )
    def combine_k(slots_hbm, o_hbm, y_hbm, idx_v, rows_v, y_v, sem):
        wid = lax.axis_index("s") * NC + lax.axis_index("c")
        base_t = wid * (N // NW)

        def chunk(j, carry):
            t0 = base_t + j * _CT
            pltpu.sync_copy(slots_hbm.at[pl.ds(TOPK * t0, TOPK * _CT)], idx_v)
            pltpu.async_copy(o_hbm.at[idx_v], rows_v, sem).wait()

            def per_tok(i, c1):
                def per_lane(c, c2):
                    a = rows_v[2 * i, pl.ds(c * 16, 16)]
                    b = rows_v[2 * i + 1, pl.ds(c * 16, 16)]
                    y_v[i, pl.ds(c * 16, 16)] = a + b
                    return c2
                return lax.fori_loop(0, H // 16, per_lane, c1)

            lax.fori_loop(0, _CT, per_tok, 0)
            pltpu.sync_copy(y_v, y_hbm.at[pl.ds(t0, _CT)])
            return carry

        lax.fori_loop(0, (N // NW) // _CT, chunk, 0)

    return combine_k


def _sc_combine(slot_of_assign, o_scaled):
    return _make_sc_combine()(slot_of_assign, o_scaled)


def _dispatch_indices(top_idx, rw):
    e_flat = top_idx.reshape(-1).astype(jnp.int32)
    w_flat = rw.reshape(-1)
    order = jnp.argsort(e_flat)
    sorted_e = e_flat[order]
    g = jnp.bincount(e_flat, length=E).astype(jnp.int32)
    gpad = ((g + T - 1) // T) * T
    padded_end = jnp.cumsum(gpad).astype(jnp.int32)
    padded_start = padded_end - gpad
    group_start = (jnp.cumsum(g) - g).astype(jnp.int32)
    p = jnp.arange(A, dtype=jnp.int32)
    slot = padded_start[sorted_e] + (p - group_start[sorted_e])
    token_of_slot = jnp.zeros((P,), jnp.int32).at[slot].set(
        (order // TOPK).astype(jnp.int32))
    slot_w = jnp.zeros((P,), jnp.float32).at[slot].set(w_flat[order])
    slot_of_assign = jnp.zeros((A,), jnp.int32).at[order].set(slot)
    tile_rows = jnp.arange(NT, dtype=jnp.int32) * T
    expert_of_tile = jnp.minimum(
        jnp.searchsorted(padded_end, tile_rows, side="right"), E - 1
    ).astype(jnp.int32)
    n_active = (padded_end[-1] // T).astype(jnp.int32).reshape(1)
    return token_of_slot, slot_w, slot_of_assign, expert_of_tile, n_active


def kernel(x, gate_w, gate_b, fc1_w, fc1_b, fc2_w, fc2_b):
    Bs, Ss, Hd = x.shape
    xf = x.reshape(-1, Hd)
    top_idx, rw = _router(xf, gate_w, gate_b)
    token_of_slot, slot_w, slot_of_assign, expert_of_tile, n_active = (
        _dispatch_indices(top_idx, rw))
    x_sorted = _sc_gather(token_of_slot, xf)
    o_scaled = _gmm(expert_of_tile, n_active, x_sorted, fc1_w, fc1_b,
                    fc2_w, fc2_b, slot_w.reshape(P, 1))
    y = _sc_combine(slot_of_assign, o_scaled)
    return y.reshape(Bs, Ss, Hd)


# speedup vs baseline: 7.8558x; 1.7213x over previous
import functools

import jax
import jax.numpy as jnp
from jax import lax
from jax.experimental import pallas as pl
from jax.experimental.pallas import tpu as pltpu
from jax.experimental.pallas import tpu_sc as plsc

E = 64
TOPK = 2
H = 768
F = 1024
N = 4096
A = N * TOPK
T = 128
P = A + E * T
NT = P // T

NC = 2
NS = 16
NW = NC * NS

_SQRT_HALF = 0.7071067811865476


def _gelu_exact(v):
    return 0.5 * v * (1.0 + lax.erf(v * _SQRT_HALF))


_RB = 512


def _router_body(x_ref, gw_ref, gb_ref, idx_ref, rw_ref):
    x = x_ref[...]
    logits = lax.dot_general(x, gw_ref[...], (((1,), (1,)), ((), ())),
                             preferred_element_type=jnp.float32)
    logits = logits + gb_ref[...][None, :]
    ids = lax.broadcasted_iota(jnp.int32, logits.shape, 1)
    neg = jnp.float32(jnp.finfo(jnp.float32).min)
    m1 = jnp.max(logits, axis=1, keepdims=True)
    a1 = jnp.min(jnp.where(logits == m1, ids, E), axis=1, keepdims=True)
    l2 = jnp.where(ids == a1, neg, logits)
    m2 = jnp.max(l2, axis=1, keepdims=True)
    a2 = jnp.min(jnp.where(l2 == m2, ids, E), axis=1, keepdims=True)
    t = jnp.exp(m2 - m1)
    w1 = 1.0 / (1.0 + t)
    idx_ref[...] = jnp.concatenate([a1, a2], axis=1)
    rw_ref[...] = jnp.concatenate([w1, t * w1], axis=1)


def _router(xf, gate_w, gate_b):
    return pl.pallas_call(
        _router_body,
        grid=(N // _RB,),
        in_specs=[
            pl.BlockSpec((_RB, H), lambda i: (i, 0)),
            pl.BlockSpec((E, H), lambda i: (0, 0)),
            pl.BlockSpec((E,), lambda i: (0,)),
        ],
        out_specs=[
            pl.BlockSpec((_RB, TOPK), lambda i: (i, 0)),
            pl.BlockSpec((_RB, TOPK), lambda i: (i, 0)),
        ],
        out_shape=[
            jax.ShapeDtypeStruct((N, TOPK), jnp.int32),
            jax.ShapeDtypeStruct((N, TOPK), jnp.float32),
        ],
        compiler_params=pltpu.CompilerParams(dimension_semantics=("arbitrary",)),
    )(xf, gate_w, gate_b)


_DTW = N // NW
_DHF = _DTW // 2


@functools.lru_cache(maxsize=None)
def _make_sc_dispatch():
    mesh = plsc.VectorSubcoreMesh(
        core_axis_name="c", subcore_axis_name="s",
        num_cores=NC, num_subcores=NS)

    @functools.partial(
        pl.kernel,
        out_type=jax.ShapeDtypeStruct((P, H), jnp.float32),
        mesh=mesh,
        scratch_types=[
            pltpu.VMEM((_DTW, H), jnp.float32),
            [pltpu.VMEM((_DHF,), jnp.int32) for _ in range(4)],
            pltpu.SemaphoreType.DMA,
        ],
    )
    def dispatch_k(sk0_hbm, sk1_hbm, x_hbm, out_hbm, buf, idxs, sem):
        wid = lax.axis_index("s") * NC + lax.axis_index("c")
        t0 = wid * _DTW
        pltpu.sync_copy(x_hbm.at[pl.ds(t0, _DTW)], buf)
        pltpu.sync_copy(sk0_hbm.at[pl.ds(t0, _DHF)], idxs[0])
        pltpu.sync_copy(sk0_hbm.at[pl.ds(t0 + _DHF, _DHF)], idxs[1])
        pltpu.sync_copy(sk1_hbm.at[pl.ds(t0, _DHF)], idxs[2])
        pltpu.sync_copy(sk1_hbm.at[pl.ds(t0 + _DHF, _DHF)], idxs[3])
        handles = [
            pltpu.async_copy(buf.at[pl.ds(0, _DHF)], out_hbm.at[idxs[0]], sem),
            pltpu.async_copy(buf.at[pl.ds(_DHF, _DHF)], out_hbm.at[idxs[1]], sem),
            pltpu.async_copy(buf.at[pl.ds(0, _DHF)], out_hbm.at[idxs[2]], sem),
            pltpu.async_copy(buf.at[pl.ds(_DHF, _DHF)], out_hbm.at[idxs[3]], sem),
        ]
        for h in handles:
            h.wait()

    return dispatch_k


def _sc_dispatch(slots_k0, slots_k1, xf):
    return _make_sc_dispatch()(slots_k0, slots_k1, xf)


def _ffn_body(er_ref, nr_ref, x_ref, w1_ref, b1_ref, w2_ref, b2_ref, sw_ref,
              o_ref):
    t = pl.program_id(0)

    @pl.when(t < nr_ref[0])
    def _():
        x = x_ref[...]
        h = lax.dot_general(x, w1_ref[0], (((1,), (1,)), ((), ())),
                            preferred_element_type=jnp.float32)
        h = _gelu_exact(h + b1_ref[0])
        o = lax.dot_general(h, w2_ref[0], (((1,), (1,)), ((), ())),
                            preferred_element_type=jnp.float32)
        o_ref[...] = (o + b2_ref[0]) * sw_ref[...]


def _gmm(expert_of_tile, n_active, x_sorted, fc1_w, fc1_b, fc2_w, fc2_b,
         slot_w):
    def rowblk(t, er, nr):
        return (jnp.minimum(t, nr[0] - 1), 0)

    grid_spec = pltpu.PrefetchScalarGridSpec(
        num_scalar_prefetch=2,
        grid=(NT,),
        in_specs=[
            pl.BlockSpec((T, H), rowblk),
            pl.BlockSpec((1, F, H), lambda t, er, nr: (er[t], 0, 0)),
            pl.BlockSpec((1, 1, F), lambda t, er, nr: (er[t], 0, 0)),
            pl.BlockSpec((1, H, F), lambda t, er, nr: (er[t], 0, 0)),
            pl.BlockSpec((1, 1, H), lambda t, er, nr: (er[t], 0, 0)),
            pl.BlockSpec((T, 1), rowblk),
        ],
        out_specs=pl.BlockSpec((T, H), rowblk),
    )
    return pl.pallas_call(
        _ffn_body,
        grid_spec=grid_spec,
        out_shape=jax.ShapeDtypeStruct((P, H), jnp.float32),
        compiler_params=pltpu.CompilerParams(dimension_semantics=("arbitrary",)),
    )(expert_of_tile, n_active, x_sorted, fc1_w, fc1_b.reshape(E, 1, F),
      fc2_w, fc2_b.reshape(E, 1, H), slot_w)


_CT = 32


@functools.lru_cache(maxsize=None)
def _make_sc_combine():
    mesh = plsc.VectorSubcoreMesh(
        core_axis_name="c", subcore_axis_name="s",
        num_cores=NC, num_subcores=NS)

    @functools.partial(
        pl.kernel,
        out_type=jax.ShapeDtypeStruct((N, H), jnp.float32),
        mesh=mesh,
        scratch_types=[
            pltpu.VMEM((2 * _CT,), jnp.int32),
            pltpu.VMEM((2 * _CT, H), jnp.float32),
            pltpu.VMEM((_CT, H), jnp.float32),
            pltpu.SemaphoreType.DMA,
        ],
    )
    def combine_k(slots_hbm, o_hbm, y_hbm, idx_v, rows_v, y_v, sem):
        wid = lax.axis_index("s") * NC + lax.axis_index("c")
        base_t = wid * (N // NW)

        def chunk(j, carry):
            t0 = base_t + j * _CT
            pltpu.sync_copy(slots_hbm.at[pl.ds(TOPK * t0, TOPK * _CT)], idx_v)
            pltpu.async_copy(o_hbm.at[idx_v], rows_v, sem).wait()

            def per_tok(i, c1):
                def per_lane(c, c2):
                    a = rows_v[2 * i, pl.ds(c * 16, 16)]
                    b = rows_v[2 * i + 1, pl.ds(c * 16, 16)]
                    y_v[i, pl.ds(c * 16, 16)] = a + b
                    return c2
                return lax.fori_loop(0, H // 16, per_lane, c1)

            lax.fori_loop(0, _CT, per_tok, 0)
            pltpu.sync_copy(y_v, y_hbm.at[pl.ds(t0, _CT)])
            return carry

        lax.fori_loop(0, (N // NW) // _CT, chunk, 0)

    return combine_k


def _sc_combine(slot_of_assign, o_scaled):
    return _make_sc_combine()(slot_of_assign, o_scaled)


def _dispatch_indices(top_idx, rw):
    e_flat = top_idx.reshape(-1).astype(jnp.int32)
    w_flat = rw.reshape(-1)
    order = jnp.argsort(e_flat)
    sorted_e = e_flat[order]
    g = jnp.bincount(e_flat, length=E).astype(jnp.int32)
    gpad = ((g + T - 1) // T) * T
    padded_end = jnp.cumsum(gpad).astype(jnp.int32)
    padded_start = padded_end - gpad
    group_start = (jnp.cumsum(g) - g).astype(jnp.int32)
    p = jnp.arange(A, dtype=jnp.int32)
    slot = padded_start[sorted_e] + (p - group_start[sorted_e])
    slot_w = jnp.zeros((P,), jnp.float32).at[slot].set(w_flat[order])
    slot_of_assign = jnp.zeros((A,), jnp.int32).at[order].set(slot)
    tile_rows = jnp.arange(NT, dtype=jnp.int32) * T
    expert_of_tile = jnp.minimum(
        jnp.searchsorted(padded_end, tile_rows, side="right"), E - 1
    ).astype(jnp.int32)
    n_active = (padded_end[-1] // T).astype(jnp.int32).reshape(1)
    return slot_w, slot_of_assign, expert_of_tile, n_active


def kernel(x, gate_w, gate_b, fc1_w, fc1_b, fc2_w, fc2_b):
    Bs, Ss, Hd = x.shape
    xf = x.reshape(-1, Hd)
    top_idx, rw = _router(xf, gate_w, gate_b)
    slot_w, slot_of_assign, expert_of_tile, n_active = (
        _dispatch_indices(top_idx, rw))
    slot_tk = slot_of_assign.reshape(N, TOPK)
    x_sorted = _sc_dispatch(slot_tk[:, 0], slot_tk[:, 1], xf)
    o_scaled = _gmm(expert_of_tile, n_active, x_sorted, fc1_w, fc1_b,
                    fc2_w, fc2_b, slot_w.reshape(P, 1))
    y = _sc_combine(slot_of_assign, o_scaled)
    return y.reshape(Bs, Ss, Hd)
